# baseline (speedup 1.0000x reference)
import functools

import jax
import jax.numpy as jnp
from jax import lax
from jax.experimental import pallas as pl
from jax.experimental.pallas import tpu as pltpu
from jax.experimental.pallas import tpu_sc as plsc

B = 64
S = 512
D = 1024
G = 16
TOP_K = 4
EPS = 1e-06
SCALE = D ** -0.5
NEG = float(jnp.finfo(jnp.float32).min)

_SC_MESH = plsc.VectorSubcoreMesh(core_axis_name="c", subcore_axis_name="s")
_GW = 8


def _dot_t(a, b):
    return lax.dot_general(a, b, (((1,), (1,)), ((), ())),
                           preferred_element_type=jnp.float32)


def _summarize_body(h_ref, w_ref, out_ref):
    h = h_ref[0]
    var = jnp.mean(h * h, axis=-1, keepdims=True)
    normed = w_ref[...] * (h * lax.rsqrt(var + EPS))
    out_ref[0] = jnp.mean(normed, axis=0, keepdims=True)


_CW = 128


@functools.partial(
    pl.kernel,
    out_type=jax.ShapeDtypeStruct((G, D), jnp.float32),
    mesh=_SC_MESH,
    scratch_types=[
        pltpu.VMEM((B,), jnp.int32),
        pltpu.VMEM((B, _CW), jnp.float32),
        pltpu.VMEM((G, _CW), jnp.float32),
    ],
    compiler_params=pltpu.CompilerParams(needs_layout_passes=False),
)
def _sc_segsum(ps_hbm, gid_hbm, out_hbm, idx_v, rows_v, acc_v):
    cid = lax.axis_index("c")
    sid = lax.axis_index("s")
    wid = sid * 2 + cid
    col0 = wid * _CW
    lane = lax.iota(jnp.int32, 16)

    @pl.when(wid < D // _CW)
    def _():
        _sc_segsum_worker(ps_hbm, gid_hbm, out_hbm, idx_v, rows_v, acc_v,
                          col0, lane)


def _sc_segsum_worker(ps_hbm, gid_hbm, out_hbm, idx_v, rows_v, acc_v,
                      col0, lane):
    pltpu.sync_copy(gid_hbm, idx_v)
    pltpu.sync_copy(ps_hbm.at[:, pl.ds(col0, _CW)], rows_v)
    for g in range(G):
        for j in range(_CW // 16):
            acc_v[g, pl.ds(j * 16, 16)] = jnp.zeros((16,), jnp.float32)

    def body(b, _):
        bb = jnp.full((16,), b, jnp.int32)
        gid = plsc.load_gather(idx_v, [bb])
        for j in range(_CW // 16):
            col = j * 16 + lane
            chunk = plsc.load_gather(rows_v, [bb, col])
            plsc.addupdate_scatter(acc_v, [gid, col], chunk)
        return _

    lax.fori_loop(0, B, body, 0)
    pltpu.sync_copy(acc_v, out_hbm.at[:, pl.ds(col0, _CW)])


def _attention_body(sums_ref, gid_ref, lnw_ref, wq_ref, wk_ref, wv_ref,
                    wo_ref, wb_ref, wgc_ref, bg_ref,
                    attn_ref, zcct_ref):
    sums = sums_ref[...]
    gids = gid_ref[...]
    rows = lax.broadcasted_iota(jnp.int32, (G, B), 0)
    onehot = (rows == gids).astype(jnp.float32)
    counts = jnp.sum(onehot, axis=1, keepdims=True)
    summaries = sums / jnp.maximum(counts, 1.0)

    var = jnp.mean(summaries * summaries, axis=-1, keepdims=True)
    normed = lnw_ref[...] * (summaries * lax.rsqrt(var + EPS))

    q = _dot_t(normed, wq_ref[...])
    k = _dot_t(normed, wk_ref[...])
    v = _dot_t(normed, wv_ref[...])
    scores = _dot_t(q, k) * SCALE

    col = lax.broadcasted_iota(jnp.int32, (G, G), 1)
    mask = lax.broadcasted_iota(jnp.int32, (G, G), 0) == col
    work = scores
    for _ in range(TOP_K):
        m = jnp.max(work, axis=1, keepdims=True)
        is_max = work == m
        first = jnp.min(jnp.where(is_max, col, G), axis=1, keepdims=True)
        sel = col == first
        mask = mask | sel
        work = jnp.where(sel, NEG, work)

    masked = jnp.where(mask, scores, NEG)
    mx = jnp.max(masked, axis=1, keepdims=True)
    e = jnp.exp(masked - mx)
    attn = e / jnp.sum(e, axis=1, keepdims=True)
    attn_ref[...] = attn

    cross = jnp.dot(attn, v, preferred_element_type=jnp.float32)
    cross = _dot_t(cross, wo_ref[...])
    ct = _dot_t(cross, wb_ref[...])
    zc = _dot_t(ct, wgc_ref[...]) + bg_ref[...]
    zcct_ref[:, :D] = zc
    zcct_ref[:, D:] = ct


@functools.partial(
    pl.kernel,
    out_type=jax.ShapeDtypeStruct((B, 2 * D), jnp.float32),
    mesh=_SC_MESH,
    scratch_types=[
        pltpu.VMEM((B // _GW,), jnp.int32),
        pltpu.VMEM((B // _GW, 2 * D), jnp.float32),
        pltpu.SemaphoreType.DMA,
    ],
)
def _sc_gather(table_hbm, gid_hbm, out_hbm, idx_v, rows_v, sem):
    cid = lax.axis_index("c")
    sid = lax.axis_index("s")
    n = B // _GW

    @pl.when(jnp.logical_and(cid == 0, sid < _GW))
    def _():
        base = sid * n
        pltpu.sync_copy(gid_hbm.at[pl.ds(base, n)], idx_v)
        pltpu.async_copy(table_hbm.at[idx_v], rows_v, sem).wait()
        pltpu.sync_copy(rows_v, out_hbm.at[pl.ds(base, n)])


def _fuse_body(h_ref, wgh_ref, zcct_ref, out_ref):
    h = h_ref[0]
    row = zcct_ref[0]
    z = lax.dot_general(h.astype(jnp.bfloat16), wgh_ref[...],
                        (((1,), (1,)), ((), ())),
                        preferred_element_type=jnp.float32)
    z = z + row[:, :D]
    gate = 1.0 / (1.0 + jnp.exp(-z))
    out_ref[0] = h + gate * row[:, D:]


def kernel(hidden_states, group_ids, ln_summary_w, ln_cross_w, Wq, Wk, Wv,
           Wo, W_gate, b_gate, W_broadcast):
    gids = group_ids.astype(jnp.int32)
    lnw_s = ln_summary_w.reshape(1, D)
    lnw_c = ln_cross_w.reshape(1, D)
    wg_h = W_gate[:, :D].astype(jnp.bfloat16)
    wg_c = W_gate[:, D:]
    bg = b_gate.reshape(1, D)

    per_series = pl.pallas_call(
        _summarize_body,
        grid=(B,),
        in_specs=[
            pl.BlockSpec((1, S, D), lambda b: (b, 0, 0)),
            pl.BlockSpec((1, D), lambda b: (0, 0)),
        ],
        out_specs=pl.BlockSpec((1, 1, D), lambda b: (b, 0, 0)),
        out_shape=jax.ShapeDtypeStruct((B, 1, D), jnp.float32),
    )(hidden_states, lnw_s)
    per_series = per_series.reshape(B, D)

    sums = _sc_segsum(per_series, gids)

    attn, zcct_g = pl.pallas_call(
        _attention_body,
        out_shape=(
            jax.ShapeDtypeStruct((G, G), jnp.float32),
            jax.ShapeDtypeStruct((G, 2 * D), jnp.float32),
        ),
    )(sums, gids.reshape(1, B), lnw_c, Wq, Wk, Wv, Wo,
      W_broadcast, wg_c, bg)

    zcct = _sc_gather(zcct_g, gids).reshape(B, 1, 2 * D)

    out = pl.pallas_call(
        _fuse_body,
        grid=(B,),
        in_specs=[
            pl.BlockSpec((1, S, D), lambda b: (b, 0, 0)),
            pl.BlockSpec((D, D), lambda b: (0, 0)),
            pl.BlockSpec((1, 1, 2 * D), lambda b: (b, 0, 0)),
        ],
        out_specs=pl.BlockSpec((1, S, D), lambda b: (b, 0, 0)),
        out_shape=jax.ShapeDtypeStruct((B, S, D), jnp.float32),
    )(hidden_states, wg_h, zcct)

    return (out, attn)


# speedup vs baseline: 1.1056x; 1.1056x over previous
import functools

import jax
import jax.numpy as jnp
from jax import lax
from jax.experimental import pallas as pl
from jax.experimental.pallas import tpu as pltpu
from jax.experimental.pallas import tpu_sc as plsc

B = 64
S = 512
D = 1024
G = 16
TOP_K = 4
EPS = 1e-06
SCALE = D ** -0.5
NEG = float(jnp.finfo(jnp.float32).min)

_SC_MESH = plsc.VectorSubcoreMesh(core_axis_name="c", subcore_axis_name="s")
_GW = 8


def _dot_t(a, b):
    return lax.dot_general(a, b, (((1,), (1,)), ((), ())),
                           preferred_element_type=jnp.float32)


def _summarize_body(h_ref, w_ref, out_ref):
    h = h_ref[...]
    var = jnp.mean(h * h, axis=-1, keepdims=True)
    normed = w_ref[...] * (h * lax.rsqrt(var + EPS))
    out_ref[:, 0, :] = jnp.mean(normed, axis=1)


_CW = 128


@functools.partial(
    pl.kernel,
    out_type=jax.ShapeDtypeStruct((G, D), jnp.float32),
    mesh=_SC_MESH,
    scratch_types=[
        pltpu.VMEM((B,), jnp.int32),
        pltpu.VMEM((B, _CW), jnp.float32),
        pltpu.VMEM((G, _CW), jnp.float32),
    ],
    compiler_params=pltpu.CompilerParams(needs_layout_passes=False),
)
def _sc_segsum(ps_hbm, gid_hbm, out_hbm, idx_v, rows_v, acc_v):
    cid = lax.axis_index("c")
    sid = lax.axis_index("s")
    wid = sid * 2 + cid
    col0 = wid * _CW
    lane = lax.iota(jnp.int32, 16)

    @pl.when(wid < D // _CW)
    def _():
        _sc_segsum_worker(ps_hbm, gid_hbm, out_hbm, idx_v, rows_v, acc_v,
                          col0, lane)


def _sc_segsum_worker(ps_hbm, gid_hbm, out_hbm, idx_v, rows_v, acc_v,
                      col0, lane):
    pltpu.sync_copy(gid_hbm, idx_v)
    pltpu.sync_copy(ps_hbm.at[:, pl.ds(col0, _CW)], rows_v)
    for g in range(G):
        for j in range(_CW // 16):
            acc_v[g, pl.ds(j * 16, 16)] = jnp.zeros((16,), jnp.float32)

    def body(b, _):
        bb = jnp.full((16,), b, jnp.int32)
        gid = plsc.load_gather(idx_v, [bb])
        for j in range(_CW // 16):
            col = j * 16 + lane
            chunk = plsc.load_gather(rows_v, [bb, col])
            plsc.addupdate_scatter(acc_v, [gid, col], chunk)
        return _

    lax.fori_loop(0, B, body, 0)
    pltpu.sync_copy(acc_v, out_hbm.at[:, pl.ds(col0, _CW)])


def _attention_body(sums_ref, gid_ref, lnw_ref, wq_ref, wk_ref, wv_ref,
                    wo_ref, wb_ref, wgc_ref, bg_ref,
                    attn_ref, zcct_ref):
    sums = sums_ref[...]
    gids = gid_ref[...]
    rows = lax.broadcasted_iota(jnp.int32, (G, B), 0)
    onehot = (rows == gids).astype(jnp.float32)
    counts = jnp.sum(onehot, axis=1, keepdims=True)
    summaries = sums / jnp.maximum(counts, 1.0)

    var = jnp.mean(summaries * summaries, axis=-1, keepdims=True)
    normed = lnw_ref[...] * (summaries * lax.rsqrt(var + EPS))

    q = _dot_t(normed, wq_ref[...])
    k = _dot_t(normed, wk_ref[...])
    v = _dot_t(normed, wv_ref[...])
    scores = _dot_t(q, k) * SCALE

    col = lax.broadcasted_iota(jnp.int32, (G, G), 1)
    mask = lax.broadcasted_iota(jnp.int32, (G, G), 0) == col
    work = scores
    for _ in range(TOP_K):
        m = jnp.max(work, axis=1, keepdims=True)
        is_max = work == m
        first = jnp.min(jnp.where(is_max, col, G), axis=1, keepdims=True)
        sel = col == first
        mask = mask | sel
        work = jnp.where(sel, NEG, work)

    masked = jnp.where(mask, scores, NEG)
    mx = jnp.max(masked, axis=1, keepdims=True)
    e = jnp.exp(masked - mx)
    attn = e / jnp.sum(e, axis=1, keepdims=True)
    attn_ref[...] = attn

    cross = jnp.dot(attn, v, preferred_element_type=jnp.float32)
    cross = _dot_t(cross, wo_ref[...])
    ct = _dot_t(cross, wb_ref[...])
    zc = _dot_t(ct, wgc_ref[...]) + bg_ref[...]
    zcct_ref[:, :D] = zc
    zcct_ref[:, D:] = ct


@functools.partial(
    pl.kernel,
    out_type=jax.ShapeDtypeStruct((B, 2 * D), jnp.float32),
    mesh=_SC_MESH,
    scratch_types=[
        pltpu.VMEM((B // _GW,), jnp.int32),
        pltpu.VMEM((B // _GW, 2 * D), jnp.float32),
        pltpu.SemaphoreType.DMA,
    ],
)
def _sc_gather(table_hbm, gid_hbm, out_hbm, idx_v, rows_v, sem):
    cid = lax.axis_index("c")
    sid = lax.axis_index("s")
    n = B // _GW

    @pl.when(jnp.logical_and(cid == 0, sid < _GW))
    def _():
        base = sid * n
        pltpu.sync_copy(gid_hbm.at[pl.ds(base, n)], idx_v)
        pltpu.async_copy(table_hbm.at[idx_v], rows_v, sem).wait()
        pltpu.sync_copy(rows_v, out_hbm.at[pl.ds(base, n)])


def _fuse_body(h_ref, wgh_ref, zcct_ref, out_ref):
    h = h_ref[0]
    row = zcct_ref[0]
    z = lax.dot_general(h.astype(jnp.bfloat16), wgh_ref[...],
                        (((1,), (1,)), ((), ())),
                        preferred_element_type=jnp.float32)
    z = z + row[:, :D]
    gate = 1.0 / (1.0 + jnp.exp(-z))
    out_ref[0] = h + gate * row[:, D:]


def kernel(hidden_states, group_ids, ln_summary_w, ln_cross_w, Wq, Wk, Wv,
           Wo, W_gate, b_gate, W_broadcast):
    gids = group_ids.astype(jnp.int32)
    lnw_s = ln_summary_w.reshape(1, D)
    lnw_c = ln_cross_w.reshape(1, D)
    wg_h = W_gate[:, :D].astype(jnp.bfloat16)
    wg_c = W_gate[:, D:]
    bg = b_gate.reshape(1, D)

    per_series = pl.pallas_call(
        _summarize_body,
        grid=(B // 4,),
        in_specs=[
            pl.BlockSpec((4, S, D), lambda b: (b, 0, 0)),
            pl.BlockSpec((1, D), lambda b: (0, 0)),
        ],
        out_specs=pl.BlockSpec((4, 1, D), lambda b: (b, 0, 0)),
        out_shape=jax.ShapeDtypeStruct((B, 1, D), jnp.float32),
    )(hidden_states, lnw_s)
    per_series = per_series.reshape(B, D)

    sums = _sc_segsum(per_series, gids)

    attn, zcct_g = pl.pallas_call(
        _attention_body,
        out_shape=(
            jax.ShapeDtypeStruct((G, G), jnp.float32),
            jax.ShapeDtypeStruct((G, 2 * D), jnp.float32),
        ),
    )(sums, gids.reshape(1, B), lnw_c, Wq, Wk, Wv, Wo,
      W_broadcast, wg_c, bg)

    zcct = _sc_gather(zcct_g, gids).reshape(B, 1, 2 * D)

    out = pl.pallas_call(
        _fuse_body,
        grid=(B,),
        in_specs=[
            pl.BlockSpec((1, S, D), lambda b: (b, 0, 0)),
            pl.BlockSpec((D, D), lambda b: (0, 0)),
            pl.BlockSpec((1, 1, 2 * D), lambda b: (b, 0, 0)),
        ],
        out_specs=pl.BlockSpec((1, S, D), lambda b: (b, 0, 0)),
        out_shape=jax.ShapeDtypeStruct((B, S, D), jnp.float32),
    )(hidden_states, wg_h, zcct)

    return (out, attn)


# speedup vs baseline: 1.2012x; 1.0865x over previous
import functools

import jax
import jax.numpy as jnp
from jax import lax
from jax.experimental import pallas as pl
from jax.experimental.pallas import tpu as pltpu
from jax.experimental.pallas import tpu_sc as plsc

B = 64
S = 512
D = 1024
G = 16
TOP_K = 4
EPS = 1e-06
SCALE = D ** -0.5
NEG = float(jnp.finfo(jnp.float32).min)

_SC_MESH = plsc.VectorSubcoreMesh(core_axis_name="c", subcore_axis_name="s")
_GW = 8


def _dot_t(a, b):
    return lax.dot_general(a, b, (((1,), (1,)), ((), ())),
                           preferred_element_type=jnp.float32)


def _summarize_body(h_ref, w_ref, out_ref):
    h = h_ref[...]
    var = jnp.mean(h * h, axis=-1, keepdims=True)
    normed = w_ref[...] * (h * lax.rsqrt(var + EPS))
    out_ref[:, 0, :] = jnp.mean(normed, axis=1)


_CW = 128


@functools.partial(
    pl.kernel,
    out_type=jax.ShapeDtypeStruct((G, D), jnp.float32),
    mesh=_SC_MESH,
    scratch_types=[
        pltpu.VMEM((B,), jnp.int32),
        pltpu.VMEM((B, _CW), jnp.float32),
        pltpu.VMEM((G, _CW), jnp.float32),
    ],
    compiler_params=pltpu.CompilerParams(needs_layout_passes=False),
)
def _sc_segsum(ps_hbm, gid_hbm, out_hbm, idx_v, rows_v, acc_v):
    cid = lax.axis_index("c")
    sid = lax.axis_index("s")
    wid = sid * 2 + cid
    col0 = wid * _CW
    lane = lax.iota(jnp.int32, 16)

    @pl.when(wid < D // _CW)
    def _():
        _sc_segsum_worker(ps_hbm, gid_hbm, out_hbm, idx_v, rows_v, acc_v,
                          col0, lane)


def _sc_segsum_worker(ps_hbm, gid_hbm, out_hbm, idx_v, rows_v, acc_v,
                      col0, lane):
    pltpu.sync_copy(gid_hbm, idx_v)
    pltpu.sync_copy(ps_hbm.at[:, pl.ds(col0, _CW)], rows_v)
    for g in range(G):
        for j in range(_CW // 16):
            acc_v[g, pl.ds(j * 16, 16)] = jnp.zeros((16,), jnp.float32)

    def body(b, _):
        bb = jnp.full((16,), b, jnp.int32)
        gid = plsc.load_gather(idx_v, [bb])
        for j in range(_CW // 16):
            col = j * 16 + lane
            chunk = plsc.load_gather(rows_v, [bb, col])
            plsc.addupdate_scatter(acc_v, [gid, col], chunk)
        return _

    lax.fori_loop(0, B, body, 0)
    pltpu.sync_copy(acc_v, out_hbm.at[:, pl.ds(col0, _CW)])


def _attention_body(sums_ref, gid_ref, lnw_ref, wq_ref, wk_ref, wv_ref,
                    wo_ref, wb_ref, wgc_ref, bg_ref,
                    attn_ref, zcct_ref):
    sums = sums_ref[...]
    gids = gid_ref[...]
    rows = lax.broadcasted_iota(jnp.int32, (G, B), 0)
    onehot = (rows == gids).astype(jnp.float32)
    counts = jnp.sum(onehot, axis=1, keepdims=True)
    summaries = sums / jnp.maximum(counts, 1.0)

    var = jnp.mean(summaries * summaries, axis=-1, keepdims=True)
    normed = lnw_ref[...] * (summaries * lax.rsqrt(var + EPS))

    q = _dot_t(normed, wq_ref[...])
    k = _dot_t(normed, wk_ref[...])
    v = _dot_t(normed, wv_ref[...])
    scores = _dot_t(q, k) * SCALE

    col = lax.broadcasted_iota(jnp.int32, (G, G), 1)
    mask = lax.broadcasted_iota(jnp.int32, (G, G), 0) == col
    work = scores
    for _ in range(TOP_K):
        m = jnp.max(work, axis=1, keepdims=True)
        is_max = work == m
        first = jnp.min(jnp.where(is_max, col, G), axis=1, keepdims=True)
        sel = col == first
        mask = mask | sel
        work = jnp.where(sel, NEG, work)

    masked = jnp.where(mask, scores, NEG)
    mx = jnp.max(masked, axis=1, keepdims=True)
    e = jnp.exp(masked - mx)
    attn = e / jnp.sum(e, axis=1, keepdims=True)
    attn_ref[...] = attn

    cross = jnp.dot(attn, v, preferred_element_type=jnp.float32)
    cross = _dot_t(cross, wo_ref[...])
    ct = _dot_t(cross, wb_ref[...])
    zc = _dot_t(ct, wgc_ref[...]) + bg_ref[...]
    zcct_ref[:, :D] = zc
    zcct_ref[:, D:] = ct


@functools.partial(
    pl.kernel,
    out_type=jax.ShapeDtypeStruct((B, 2 * D), jnp.float32),
    mesh=_SC_MESH,
    scratch_types=[
        pltpu.VMEM((B // _GW,), jnp.int32),
        pltpu.VMEM((B // _GW, 2 * D), jnp.float32),
        pltpu.SemaphoreType.DMA,
    ],
)
def _sc_gather(table_hbm, gid_hbm, out_hbm, idx_v, rows_v, sem):
    cid = lax.axis_index("c")
    sid = lax.axis_index("s")
    n = B // _GW

    @pl.when(jnp.logical_and(cid == 0, sid < _GW))
    def _():
        base = sid * n
        pltpu.sync_copy(gid_hbm.at[pl.ds(base, n)], idx_v)
        pltpu.async_copy(table_hbm.at[idx_v], rows_v, sem).wait()
        pltpu.sync_copy(rows_v, out_hbm.at[pl.ds(base, n)])


def _fuse_body(h_ref, wgh_ref, zcct_ref, out_ref):
    for i in range(2):
        h = h_ref[i]
        row = zcct_ref[i]
        z = lax.dot_general(h.astype(jnp.bfloat16), wgh_ref[...],
                            (((1,), (1,)), ((), ())),
                            preferred_element_type=jnp.float32)
        z = z + row[:, :D]
        gate = 1.0 / (1.0 + jnp.exp(-z))
        out_ref[i] = h + gate * row[:, D:]


def kernel(hidden_states, group_ids, ln_summary_w, ln_cross_w, Wq, Wk, Wv,
           Wo, W_gate, b_gate, W_broadcast):
    gids = group_ids.astype(jnp.int32)
    lnw_s = ln_summary_w.reshape(1, D)
    lnw_c = ln_cross_w.reshape(1, D)
    wg_h = W_gate[:, :D].astype(jnp.bfloat16)
    wg_c = W_gate[:, D:]
    bg = b_gate.reshape(1, D)

    per_series = pl.pallas_call(
        _summarize_body,
        grid=(B // 4,),
        in_specs=[
            pl.BlockSpec((4, S, D), lambda b: (b, 0, 0)),
            pl.BlockSpec((1, D), lambda b: (0, 0)),
        ],
        out_specs=pl.BlockSpec((4, 1, D), lambda b: (b, 0, 0)),
        out_shape=jax.ShapeDtypeStruct((B, 1, D), jnp.float32),
    )(hidden_states, lnw_s)
    per_series = per_series.reshape(B, D)

    sums = _sc_segsum(per_series, gids)

    attn, zcct_g = pl.pallas_call(
        _attention_body,
        out_shape=(
            jax.ShapeDtypeStruct((G, G), jnp.float32),
            jax.ShapeDtypeStruct((G, 2 * D), jnp.float32),
        ),
    )(sums, gids.reshape(1, B), lnw_c, Wq, Wk, Wv, Wo,
      W_broadcast, wg_c, bg)

    zcct = _sc_gather(zcct_g, gids).reshape(B, 1, 2 * D)

    out = pl.pallas_call(
        _fuse_body,
        grid=(B // 2,),
        in_specs=[
            pl.BlockSpec((2, S, D), lambda b: (b, 0, 0)),
            pl.BlockSpec((D, D), lambda b: (0, 0)),
            pl.BlockSpec((2, 1, 2 * D), lambda b: (b, 0, 0)),
        ],
        out_specs=pl.BlockSpec((2, S, D), lambda b: (b, 0, 0)),
        out_shape=jax.ShapeDtypeStruct((B, S, D), jnp.float32),
    )(hidden_states, wg_h, zcct)

    return (out, attn)


# speedup vs baseline: 1.2639x; 1.0522x over previous
import functools

import jax
import jax.numpy as jnp
from jax import lax
from jax.experimental import pallas as pl
from jax.experimental.pallas import tpu as pltpu
from jax.experimental.pallas import tpu_sc as plsc

B = 64
S = 512
D = 1024
G = 16
TOP_K = 4
EPS = 1e-06
SCALE = D ** -0.5
NEG = float(jnp.finfo(jnp.float32).min)

_SC_MESH = plsc.VectorSubcoreMesh(core_axis_name="c", subcore_axis_name="s")
_GW = 8


def _dot_t(a, b):
    return lax.dot_general(a, b, (((1,), (1,)), ((), ())),
                           preferred_element_type=jnp.float32)


def _summarize_body(h_ref, w_ref, out_ref):
    h = h_ref[...]
    var = jnp.mean(h * h, axis=-1, keepdims=True)
    normed = w_ref[...] * (h * lax.rsqrt(var + EPS))
    out_ref[:, 0, :] = jnp.mean(normed, axis=1)


_CW = 128


@functools.partial(
    pl.kernel,
    out_type=jax.ShapeDtypeStruct((G, D), jnp.float32),
    mesh=_SC_MESH,
    scratch_types=[
        pltpu.VMEM((B,), jnp.int32),
        pltpu.VMEM((B, _CW), jnp.float32),
        pltpu.VMEM((G, _CW), jnp.float32),
    ],
    compiler_params=pltpu.CompilerParams(needs_layout_passes=False),
)
def _sc_segsum(ps_hbm, gid_hbm, out_hbm, idx_v, rows_v, acc_v):
    cid = lax.axis_index("c")
    sid = lax.axis_index("s")
    wid = sid * 2 + cid
    col0 = wid * _CW
    lane = lax.iota(jnp.int32, 16)

    @pl.when(wid < D // _CW)
    def _():
        _sc_segsum_worker(ps_hbm, gid_hbm, out_hbm, idx_v, rows_v, acc_v,
                          col0, lane)


def _sc_segsum_worker(ps_hbm, gid_hbm, out_hbm, idx_v, rows_v, acc_v,
                      col0, lane):
    pltpu.sync_copy(gid_hbm, idx_v)
    pltpu.sync_copy(ps_hbm.at[:, pl.ds(col0, _CW)], rows_v)
    for g in range(G):
        for j in range(_CW // 16):
            acc_v[g, pl.ds(j * 16, 16)] = jnp.zeros((16,), jnp.float32)

    def body(b, _):
        bb = jnp.full((16,), b, jnp.int32)
        gid = plsc.load_gather(idx_v, [bb])
        for j in range(_CW // 16):
            col = j * 16 + lane
            chunk = plsc.load_gather(rows_v, [bb, col])
            plsc.addupdate_scatter(acc_v, [gid, col], chunk)
        return _

    lax.fori_loop(0, B, body, 0)
    pltpu.sync_copy(acc_v, out_hbm.at[:, pl.ds(col0, _CW)])


def _attention_body(sums_ref, gid_ref, lnw_ref, wq_ref, wk_ref, wv_ref,
                    wo_ref, wb_ref, wgc_ref, bg_ref,
                    attn_ref, zcct_ref):
    sums = sums_ref[...]
    gids = gid_ref[...]
    rows = lax.broadcasted_iota(jnp.int32, (G, B), 0)
    onehot = (rows == gids).astype(jnp.float32)
    counts = jnp.sum(onehot, axis=1, keepdims=True)
    summaries = sums / jnp.maximum(counts, 1.0)

    var = jnp.mean(summaries * summaries, axis=-1, keepdims=True)
    normed = lnw_ref[...] * (summaries * lax.rsqrt(var + EPS))

    q = _dot_t(normed, wq_ref[...])
    k = _dot_t(normed, wk_ref[...])
    v = _dot_t(normed, wv_ref[...])
    scores = _dot_t(q, k) * SCALE

    col = lax.broadcasted_iota(jnp.int32, (G, G), 1)
    mask = lax.broadcasted_iota(jnp.int32, (G, G), 0) == col
    work = scores
    for _ in range(TOP_K):
        m = jnp.max(work, axis=1, keepdims=True)
        is_max = work == m
        first = jnp.min(jnp.where(is_max, col, G), axis=1, keepdims=True)
        sel = col == first
        mask = mask | sel
        work = jnp.where(sel, NEG, work)

    masked = jnp.where(mask, scores, NEG)
    mx = jnp.max(masked, axis=1, keepdims=True)
    e = jnp.exp(masked - mx)
    attn = e / jnp.sum(e, axis=1, keepdims=True)
    attn_ref[...] = attn

    cross = jnp.dot(attn, v, preferred_element_type=jnp.float32)
    cross = _dot_t(cross, wo_ref[...])
    ct = _dot_t(cross, wb_ref[...])
    zc = _dot_t(ct, wgc_ref[...]) + bg_ref[...]
    zcct_ref[:, :D] = zc
    zcct_ref[:, D:] = ct


@functools.partial(
    pl.kernel,
    out_type=jax.ShapeDtypeStruct((B, 2 * D), jnp.float32),
    mesh=_SC_MESH,
    scratch_types=[
        pltpu.VMEM((B // _GW,), jnp.int32),
        pltpu.VMEM((B // _GW, 2 * D), jnp.float32),
        pltpu.SemaphoreType.DMA,
    ],
)
def _sc_gather(table_hbm, gid_hbm, out_hbm, idx_v, rows_v, sem):
    cid = lax.axis_index("c")
    sid = lax.axis_index("s")
    n = B // _GW

    @pl.when(jnp.logical_and(cid == 0, sid < _GW))
    def _():
        base = sid * n
        pltpu.sync_copy(gid_hbm.at[pl.ds(base, n)], idx_v)
        pltpu.async_copy(table_hbm.at[idx_v], rows_v, sem).wait()
        pltpu.sync_copy(rows_v, out_hbm.at[pl.ds(base, n)])


def _fuse_body(h_ref, wgh_ref, zcct_ref, out_ref):
    for i in range(4):
        h = h_ref[i]
        row = zcct_ref[i]
        z = lax.dot_general(h.astype(jnp.bfloat16), wgh_ref[...],
                            (((1,), (1,)), ((), ())),
                            preferred_element_type=jnp.float32)
        z = z + row[:, :D]
        gate = 1.0 / (1.0 + jnp.exp(-z))
        out_ref[i] = h + gate * row[:, D:]


def kernel(hidden_states, group_ids, ln_summary_w, ln_cross_w, Wq, Wk, Wv,
           Wo, W_gate, b_gate, W_broadcast):
    gids = group_ids.astype(jnp.int32)
    lnw_s = ln_summary_w.reshape(1, D)
    lnw_c = ln_cross_w.reshape(1, D)
    wg_h = W_gate[:, :D].astype(jnp.bfloat16)
    wg_c = W_gate[:, D:]
    bg = b_gate.reshape(1, D)

    per_series = pl.pallas_call(
        _summarize_body,
        grid=(B // 8,),
        in_specs=[
            pl.BlockSpec((8, S, D), lambda b: (b, 0, 0)),
            pl.BlockSpec((1, D), lambda b: (0, 0)),
        ],
        out_specs=pl.BlockSpec((8, 1, D), lambda b: (b, 0, 0)),
        out_shape=jax.ShapeDtypeStruct((B, 1, D), jnp.float32),
    )(hidden_states, lnw_s)
    per_series = per_series.reshape(B, D)

    sums = _sc_segsum(per_series, gids)

    attn, zcct_g = pl.pallas_call(
        _attention_body,
        out_shape=(
            jax.ShapeDtypeStruct((G, G), jnp.float32),
            jax.ShapeDtypeStruct((G, 2 * D), jnp.float32),
        ),
    )(sums, gids.reshape(1, B), lnw_c, Wq, Wk, Wv, Wo,
      W_broadcast, wg_c, bg)

    zcct = _sc_gather(zcct_g, gids).reshape(B, 1, 2 * D)

    out = pl.pallas_call(
        _fuse_body,
        grid=(B // 4,),
        in_specs=[
            pl.BlockSpec((4, S, D), lambda b: (b, 0, 0)),
            pl.BlockSpec((D, D), lambda b: (0, 0)),
            pl.BlockSpec((4, 1, 2 * D), lambda b: (b, 0, 0)),
        ],
        out_specs=pl.BlockSpec((4, S, D), lambda b: (b, 0, 0)),
        out_shape=jax.ShapeDtypeStruct((B, S, D), jnp.float32),
    )(hidden_states, wg_h, zcct)

    return (out, attn)
